# baseline (speedup 1.0000x reference)
import functools

import jax
import jax.numpy as jnp
from jax import lax
from jax.experimental import layout as jlayout
from jax.experimental import pallas as pl
from jax.experimental.pallas import tpu as pltpu
from jax.experimental.pallas import tpu_sc as plsc

_NC = 2
_NS = 16
_NW = _NC * _NS
_CHUNK = 320


def kernel(words, table):
    batch, seq = words.shape
    n = batch * seq
    dim = table.shape[1]
    idx = words.reshape(n).astype(jnp.int32)

    table = jlayout.with_layout_constraint(
        table,
        jlayout.Layout(major_to_minor=(0, 1), tiling=((16,),)),
    )

    b_per_w = n // _NW
    n_chunks = b_per_w // _CHUNK

    mesh = plsc.VectorSubcoreMesh(core_axis_name="c", subcore_axis_name="s")

    @functools.partial(
        pl.kernel,
        mesh=mesh,
        out_type=jax.ShapeDtypeStruct((n, dim), table.dtype),
        scratch_types=[
            pltpu.VMEM((_CHUNK,), jnp.int32),
            pltpu.VMEM((_CHUNK,), jnp.int32),
            pltpu.VMEM((_CHUNK, dim), table.dtype),
            pltpu.VMEM((_CHUNK, dim), table.dtype),
            pltpu.SemaphoreType.DMA,
            pltpu.SemaphoreType.DMA,
            pltpu.SemaphoreType.DMA,
            pltpu.SemaphoreType.DMA,
        ],
    )
    def _gather(table_hbm, idx_hbm, out_hbm,
                i0, i1, r0, r1, sg0, sg1, sw0, sw1):
        wid = lax.axis_index("s") * _NC + lax.axis_index("c")
        base = wid * b_per_w

        def idx_src(c):
            return idx_hbm.at[pl.ds(base + c * _CHUNK, _CHUNK)]

        def out_dst(c):
            return out_hbm.at[pl.ds(base + c * _CHUNK, _CHUNK)]

        @pl.loop(0, n_chunks // 2)
        def _(m):
            c0 = 2 * m
            c1 = c0 + 1

            @pl.when(m > 0)
            def _():
                pltpu.make_async_copy(r0, out_dst(c0 - 2), sw0).wait()

            pltpu.sync_copy(idx_src(c0), i0)
            g0 = pltpu.async_copy(table_hbm.at[i0], r0, sg0)
            pltpu.sync_copy(idx_src(c1), i1)

            @pl.when(m > 0)
            def _():
                pltpu.make_async_copy(r1, out_dst(c1 - 2), sw1).wait()

            g0.wait()
            pltpu.async_copy(r0, out_dst(c0), sw0)
            pltpu.async_copy(table_hbm.at[i1], r1, sg1).wait()
            pltpu.async_copy(r1, out_dst(c1), sw1)

        pltpu.make_async_copy(r0, out_dst(n_chunks - 2), sw0).wait()
        pltpu.make_async_copy(r1, out_dst(n_chunks - 1), sw1).wait()

    return _gather(table, idx).reshape(batch, seq, dim)


# speedup vs baseline: 1.0002x; 1.0002x over previous
import functools

import jax
import jax.numpy as jnp
from jax import lax
from jax.experimental import layout as jlayout
from jax.experimental import pallas as pl
from jax.experimental.pallas import tpu as pltpu
from jax.experimental.pallas import tpu_sc as plsc

_NC = 2
_NS = 16
_NW = _NC * _NS
_CHUNK = 320


def kernel(words, table):
    batch, seq = words.shape
    n = batch * seq
    dim = table.shape[1]
    idx = words.reshape(n).astype(jnp.int32)

    table = jlayout.with_layout_constraint(
        table,
        jlayout.Layout(major_to_minor=(0, 1), tiling=((16,),)),
    )

    b_per_w = n // _NW
    n_chunks = b_per_w // _CHUNK

    mesh = plsc.VectorSubcoreMesh(core_axis_name="c", subcore_axis_name="s")

    @functools.partial(
        pl.kernel,
        mesh=mesh,
        out_type=jax.ShapeDtypeStruct((n, dim), table.dtype),
        scratch_types=[
            pltpu.VMEM((_CHUNK,), jnp.int32),
            pltpu.VMEM((_CHUNK,), jnp.int32),
            pltpu.VMEM((_CHUNK, dim), table.dtype),
            pltpu.VMEM((_CHUNK, dim), table.dtype),
            pltpu.SemaphoreType.DMA,
            pltpu.SemaphoreType.DMA,
            pltpu.SemaphoreType.DMA,
            pltpu.SemaphoreType.DMA,
        ],
    )
    def _gather(table_hbm, idx_hbm, out_hbm,
                i0, i1, r0, r1, sg0, sg1, sw0, sw1):
        wid = lax.axis_index("s") * _NC + lax.axis_index("c")
        base = wid * b_per_w

        def idx_src(c):
            return idx_hbm.at[pl.ds(base + c * _CHUNK, _CHUNK)]

        def out_dst(c):
            return out_hbm.at[pl.ds(base + c * _CHUNK, _CHUNK)]

        @pl.loop(0, n_chunks // 2)
        def _(m):
            c0 = 2 * m
            c1 = c0 + 1

            @pl.when(m > 0)
            def _():
                pltpu.make_async_copy(r0, out_dst(c0 - 2), sw0).wait()

            pltpu.sync_copy(idx_src(c0), i0)
            g0 = pltpu.async_copy(table_hbm.at[i0], r0, sg0)
            pltpu.sync_copy(idx_src(c1), i1)

            @pl.when(m > 0)
            def _():
                pltpu.make_async_copy(r1, out_dst(c1 - 2), sw1).wait()

            g1 = pltpu.async_copy(table_hbm.at[i1], r1, sg1)
            g0.wait()
            pltpu.async_copy(r0, out_dst(c0), sw0)
            g1.wait()
            pltpu.async_copy(r1, out_dst(c1), sw1)

        pltpu.make_async_copy(r0, out_dst(n_chunks - 2), sw0).wait()
        pltpu.make_async_copy(r1, out_dst(n_chunks - 1), sw1).wait()

    return _gather(table, idx).reshape(batch, seq, dim)


# speedup vs baseline: 1.0017x; 1.0014x over previous
import functools

import jax
import jax.numpy as jnp
from jax import lax
from jax.experimental import layout as jlayout
from jax.experimental import pallas as pl
from jax.experimental.pallas import tpu as pltpu
from jax.experimental.pallas import tpu_sc as plsc

_NC = 2
_NS = 16
_NW = _NC * _NS
_CHUNK = 320


def kernel(words, table):
    batch, seq = words.shape
    n = batch * seq
    dim = table.shape[1]
    idx = words.reshape(n).astype(jnp.int32)

    table = jlayout.with_layout_constraint(
        table,
        jlayout.Layout(major_to_minor=(0, 1), tiling=((16,),)),
    )

    b_per_w = n // _NW
    n_chunks = b_per_w // _CHUNK

    mesh = plsc.VectorSubcoreMesh(core_axis_name="c", subcore_axis_name="s")

    @functools.partial(
        pl.kernel,
        mesh=mesh,
        out_type=jax.ShapeDtypeStruct((n, dim), table.dtype),
        scratch_types=[
            pltpu.VMEM((_CHUNK,), jnp.int32),
            pltpu.VMEM((_CHUNK,), jnp.int32),
            pltpu.VMEM((_CHUNK, dim), table.dtype),
            pltpu.VMEM((_CHUNK, dim), table.dtype),
            pltpu.SemaphoreType.DMA,
            pltpu.SemaphoreType.DMA,
            pltpu.SemaphoreType.DMA,
            pltpu.SemaphoreType.DMA,
        ],
    )
    def _gather(table_hbm, idx_hbm, out_hbm,
                i0, i1, r0, r1, sg0, sg1, sw0, sw1):
        wid = lax.axis_index("s") * _NC + lax.axis_index("c")
        base = wid * b_per_w

        def idx_src(c):
            return idx_hbm.at[pl.ds(base + c * _CHUNK, _CHUNK)]

        def out_dst(c):
            return out_hbm.at[pl.ds(base + c * _CHUNK, _CHUNK)]

        @pl.loop(0, n_chunks // 2)
        def _(m):
            c0 = 2 * m
            c1 = c0 + 1

            @pl.when(m > 0)
            def _():
                pltpu.make_async_copy(r0, out_dst(c0 - 2), sw0).wait()

            pltpu.sync_copy(idx_src(c0), i0)
            g0 = pltpu.async_copy(table_hbm.at[i0], r0, sg0)
            pltpu.sync_copy(idx_src(c1), i1)

            @pl.when(m > 0)
            def _():
                pltpu.make_async_copy(r1, out_dst(c1 - 2), sw1).wait()

            g0.wait()
            pltpu.async_copy(r0, out_dst(c0), sw0)
            pltpu.async_copy(table_hbm.at[i1], r1, sg1).wait()
            pltpu.async_copy(r1, out_dst(c1), sw1)

        pltpu.make_async_copy(r0, out_dst(n_chunks - 2), sw0).wait()
        pltpu.make_async_copy(r1, out_dst(n_chunks - 1), sw1).wait()

    return _gather(table, idx).reshape(batch, seq, dim)
